# baseline (speedup 1.0000x reference)
import functools

import jax
import jax.numpy as jnp
from jax import lax
from jax.experimental import pallas as pl
from jax.experimental.pallas import tpu as pltpu
from jax.experimental.pallas import tpu_sc as plsc

_N = 10000
_E = 320000
_D = 128
_L = 2

_NC = 2
_NS = 16
_NW = _NC * _NS
_EPW = _E // _NW
_CHUNK = 80
_NCHUNK = _EPW // _CHUNK
_RING = 4
_STRIPE = 624
_LAST_BASE = _STRIPE * (_NS - 1)
_LAST_ROWS = _N - _LAST_BASE


def _make_agg(dual):
    nchunk = _NCHUNK * 2 if dual else _NCHUNK
    mesh = plsc.VectorSubcoreMesh(core_axis_name="c", subcore_axis_name="s")

    @functools.partial(
        pl.kernel,
        mesh=mesh,
        out_type=[
            jax.ShapeDtypeStruct((_N, _D), jnp.float32),
            jax.ShapeDtypeStruct((_N, _D), jnp.float32),
        ],
        scratch_types=[
            pltpu.VMEM((_RING, 1, _CHUNK), jnp.int32),
            pltpu.VMEM((_RING, 1, _CHUNK), jnp.int32),
        ] + [pltpu.VMEM((_CHUNK, _D), jnp.float32) for _ in range(_RING)]
          + [pltpu.SemaphoreType.DMA for _ in range(2 * _RING)]
          + [pltpu.VMEM_SHARED((_N, _D), jnp.float32)],
    )
    def agg(v_hbm, w_hbm, src_hbm, dst_hbm, zeros_hbm, p0_hbm, p1_hbm, *scr):
        srcb, dstb = scr[0:2]
        bufs = scr[2:2 + _RING]
        gsems = scr[2 + _RING:2 + 2 * _RING]
        isems = scr[2 + 2 * _RING:2 + 3 * _RING]
        acc = scr[2 + 3 * _RING]
        core = lax.axis_index("c")
        sub = lax.axis_index("s")
        row = sub if dual else core * _NS + sub
        stripe = pl.multiple_of(sub * _STRIPE, 8)
        is_last = sub == _NS - 1

        def idx_issue(m, b):
            pltpu.async_copy(src_hbm.at[row, m], srcb.at[b], isems[b])
            pltpu.async_copy(dst_hbm.at[row, m], dstb.at[b], isems[b])

        def idx_wait(m, b):
            pltpu.make_async_copy(src_hbm.at[row, m], srcb.at[b],
                                  isems[b]).wait()
            pltpu.make_async_copy(dst_hbm.at[row, m], dstb.at[b],
                                  isems[b]).wait()

        def gather(b):
            @pl.when(core == 0)
            def _():
                pltpu.async_copy(v_hbm.at[srcb.at[b, 0]], bufs[b], gsems[b])

            @pl.when(core == 1)
            def _():
                pltpu.async_copy(w_hbm.at[srcb.at[b, 0]], bufs[b], gsems[b])

        def drain_scatter(b):
            @pl.when(core == 0)
            def _():
                pltpu.make_async_copy(v_hbm.at[srcb.at[b, 0]], bufs[b],
                                      gsems[b]).wait()

            @pl.when(core == 1)
            def _():
                pltpu.make_async_copy(w_hbm.at[srcb.at[b, 0]], bufs[b],
                                      gsems[b]).wait()

            pltpu.sync_copy(bufs[b], acc.at[dstb.at[b, 0]], add=True)

        for k in range(_RING - 1):
            idx_issue(k, k)
        for k in range(_RING - 2):
            idx_wait(k, k)
            gather(k)

        @pl.when(jnp.logical_not(is_last))
        def _():
            pltpu.sync_copy(zeros_hbm.at[pl.ds(0, _STRIPE)],
                            acc.at[pl.ds(stripe, _STRIPE)])

        @pl.when(is_last)
        def _():
            pltpu.sync_copy(zeros_hbm, acc.at[pl.ds(_LAST_BASE, _LAST_ROWS)])

        plsc.subcore_barrier()

        def body(i, carry):
            for u in range(_RING):
                m = i * _RING + u
                bi = (u + _RING - 1) % _RING
                bg = (u + _RING - 2) % _RING
                idx_issue(m + _RING - 1, bi)
                idx_wait(m + _RING - 2, bg)
                gather(bg)
                drain_scatter(u)
            return carry

        trips = (nchunk - _RING + 1) // _RING
        lax.fori_loop(0, trips, body, 0)
        for m in range(trips * _RING, nchunk):
            if m + _RING - 1 < nchunk:
                idx_issue(m + _RING - 1, (m + _RING - 1) % _RING)
            if m + _RING - 2 < nchunk:
                idx_wait(m + _RING - 2, (m + _RING - 2) % _RING)
                gather((m + _RING - 2) % _RING)
            drain_scatter(m % _RING)
        plsc.subcore_barrier()

        @pl.when(jnp.logical_and(core == 0, jnp.logical_not(is_last)))
        def _():
            pltpu.sync_copy(acc.at[pl.ds(stripe, _STRIPE)],
                            p0_hbm.at[pl.ds(stripe, _STRIPE)])

        @pl.when(jnp.logical_and(core == 0, is_last))
        def _():
            pltpu.sync_copy(acc.at[pl.ds(_LAST_BASE, _LAST_ROWS)],
                            p0_hbm.at[pl.ds(_LAST_BASE, _LAST_ROWS)])

        @pl.when(jnp.logical_and(core == 1, jnp.logical_not(is_last)))
        def _():
            pltpu.sync_copy(acc.at[pl.ds(stripe, _STRIPE)],
                            p1_hbm.at[pl.ds(stripe, _STRIPE)])

        @pl.when(jnp.logical_and(core == 1, is_last))
        def _():
            pltpu.sync_copy(acc.at[pl.ds(_LAST_BASE, _LAST_ROWS)],
                            p1_hbm.at[pl.ds(_LAST_BASE, _LAST_ROWS)])

    return agg


_agg1 = _make_agg(dual=False)
_agg2 = _make_agg(dual=True)

_BLK = 10000


def _tc1_body(ax, ah, h, wx, wh, b, u_out, hp_out):
    g = jnp.dot(ax[...], wx[...], preferred_element_type=jnp.float32)
    g = g + jnp.dot(ah[...], wh[...], preferred_element_type=jnp.float32)
    g = jax.nn.sigmoid(g + b[...])
    u_out[...] = g[:, _D:]
    hp_out[...] = g[:, :_D] * h[...]


def _tc1(ax, ah, h, wx, wh, b):
    row = pl.BlockSpec((_BLK, _D), lambda i: (i, 0))
    full = pl.BlockSpec((_D, 2 * _D), lambda i: (0, 0))
    bias = pl.BlockSpec((1, 2 * _D), lambda i: (0, 0))
    return pl.pallas_call(
        _tc1_body,
        grid=(_N // _BLK,),
        in_specs=[row, row, row, full, full, bias],
        out_specs=[row, row],
        out_shape=[
            jax.ShapeDtypeStruct((_N, _D), jnp.float32),
            jax.ShapeDtypeStruct((_N, _D), jnp.float32),
        ],
    )(ax, ah, h, wx, wh, b)


def _tc2_body(ax, ac0, ac1, u, h, wxc, whc, bc, out):
    acv = ac0[...] + ac1[...]
    c = jnp.dot(ax[...], wxc[...], preferred_element_type=jnp.float32)
    c = c + jnp.dot(acv, whc[...], preferred_element_type=jnp.float32)
    c = jax.nn.sigmoid(c + bc[...])
    uv = u[...]
    out[...] = uv * h[...] + (1.0 - uv) * c


def _tc2(ax, ac0, ac1, u, h, wxc, whc, bc):
    row = pl.BlockSpec((_BLK, _D), lambda i: (i, 0))
    full = pl.BlockSpec((_D, _D), lambda i: (0, 0))
    bias = pl.BlockSpec((1, _D), lambda i: (0, 0))
    return pl.pallas_call(
        _tc2_body,
        grid=(_N // _BLK,),
        in_specs=[row, row, row, row, row, full, full, bias],
        out_specs=row,
        out_shape=jax.ShapeDtypeStruct((_N, _D), jnp.float32),
    )(ax, ac0, ac1, u, h, wxc, whc, bc)


def kernel(x, hidden_states, edge_index, params):
    src1 = edge_index[0].reshape(_NW, _NCHUNK, 1, _CHUNK)
    dst1 = edge_index[1].reshape(_NW, _NCHUNK, 1, _CHUNK)
    src2 = edge_index[0].reshape(_NS, 2 * _NCHUNK, 1, _CHUNK)
    dst2 = edge_index[1].reshape(_NS, 2 * _NCHUNK, 1, _CHUNK)
    zeros = jnp.zeros((_LAST_ROWS, _D), jnp.float32)

    hiddens = []
    cur = x
    for l in range(_L):
        h = hidden_states[l]
        wr, wu, wc = params['W_r%d' % l], params['W_u%d' % l], params['W_c%d' % l]
        wx_ru = jnp.concatenate([wr[:_D], wu[:_D]], axis=1)
        wh_ru = jnp.concatenate([wr[_D:], wu[_D:]], axis=1)
        b_ru = jnp.concatenate([
            params['b_r%d' % l] + params['gb_r%d' % l],
            params['b_u%d' % l] + params['gb_u%d' % l],
        ])[None, :]
        wxc, whc = wc[:_D], wc[_D:]
        bc = (params['b_c%d' % l] + params['gb_c%d' % l])[None, :]

        ax, ah = _agg2(cur, h, src2, dst2, zeros)
        u, hp = _tc1(ax, ah, h, wx_ru, wh_ru, b_ru)
        ac0, ac1 = _agg1(hp, hp, src1, dst1, zeros)
        cur = _tc2(ax, ac0, ac1, u, h, wxc, whc, bc)
        hiddens.append(cur)
    return (cur, jnp.stack(hiddens))


# speedup vs baseline: 1.0139x; 1.0139x over previous
import functools

import jax
import jax.numpy as jnp
from jax import lax
from jax.experimental import pallas as pl
from jax.experimental.pallas import tpu as pltpu
from jax.experimental.pallas import tpu_sc as plsc

_N = 10000
_E = 320000
_D = 128
_L = 2

_NC = 2
_NS = 16
_NW = _NC * _NS
_EPW = _E // _NW
_CHUNK = 80
_NCHUNK = _EPW // _CHUNK
_RING = 4
_STRIPE = 624
_LAST_BASE = _STRIPE * (_NS - 1)
_LAST_ROWS = _N - _LAST_BASE


def _make_agg(dual):
    nchunk = _NCHUNK * 2 if dual else _NCHUNK
    mesh = plsc.VectorSubcoreMesh(core_axis_name="c", subcore_axis_name="s")

    @functools.partial(
        pl.kernel,
        mesh=mesh,
        out_type=[
            jax.ShapeDtypeStruct((_N, _D), jnp.float32),
            jax.ShapeDtypeStruct((_N, _D), jnp.float32),
        ],
        scratch_types=[
            pltpu.VMEM((_RING, 1, _CHUNK), jnp.int32),
            pltpu.VMEM((_RING, 1, _CHUNK), jnp.int32),
        ] + [pltpu.VMEM((_CHUNK, _D), jnp.float32) for _ in range(_RING)]
          + [pltpu.SemaphoreType.DMA for _ in range(2 * _RING)]
          + [pltpu.VMEM_SHARED((_N, _D), jnp.float32)],
    )
    def agg(v_hbm, w_hbm, src_hbm, dst_hbm, zeros_hbm, p0_hbm, p1_hbm, *scr):
        srcb, dstb = scr[0:2]
        bufs = scr[2:2 + _RING]
        gsems = scr[2 + _RING:2 + 2 * _RING]
        isems = scr[2 + 2 * _RING:2 + 3 * _RING]
        acc = scr[2 + 3 * _RING]
        core = lax.axis_index("c")
        sub = lax.axis_index("s")
        row = sub if dual else core * _NS + sub
        stripe = pl.multiple_of(sub * _STRIPE, 8)
        is_last = sub == _NS - 1

        def idx_issue(m, b):
            pltpu.async_copy(src_hbm.at[row, m], srcb.at[b], isems[b])
            pltpu.async_copy(dst_hbm.at[row, m], dstb.at[b], isems[b])

        def idx_wait(m, b):
            pltpu.make_async_copy(src_hbm.at[row, m], srcb.at[b],
                                  isems[b]).wait()
            pltpu.make_async_copy(dst_hbm.at[row, m], dstb.at[b],
                                  isems[b]).wait()

        def gather(b):
            @pl.when(core == 0)
            def _():
                pltpu.async_copy(v_hbm.at[srcb.at[b, 0]], bufs[b], gsems[b])

            @pl.when(core == 1)
            def _():
                pltpu.async_copy(w_hbm.at[srcb.at[b, 0]], bufs[b], gsems[b])

        def drain_scatter(b):
            @pl.when(core == 0)
            def _():
                pltpu.make_async_copy(v_hbm.at[srcb.at[b, 0]], bufs[b],
                                      gsems[b]).wait()

            @pl.when(core == 1)
            def _():
                pltpu.make_async_copy(w_hbm.at[srcb.at[b, 0]], bufs[b],
                                      gsems[b]).wait()

            pltpu.sync_copy(bufs[b], acc.at[dstb.at[b, 0]], add=True)

        for k in range(_RING - 1):
            idx_issue(k, k)
        for k in range(_RING - 2):
            idx_wait(k, k)
            gather(k)

        @pl.when(jnp.logical_not(is_last))
        def _():
            pltpu.sync_copy(zeros_hbm.at[pl.ds(0, _STRIPE)],
                            acc.at[pl.ds(stripe, _STRIPE)])

        @pl.when(is_last)
        def _():
            pltpu.sync_copy(zeros_hbm, acc.at[pl.ds(_LAST_BASE, _LAST_ROWS)])

        plsc.subcore_barrier()

        def body(i, carry):
            for u in range(_RING):
                m = i * _RING + u
                bi = (u + _RING - 1) % _RING
                bg = (u + _RING - 2) % _RING
                idx_issue(m + _RING - 1, bi)
                idx_wait(m + _RING - 2, bg)
                gather(bg)
                drain_scatter(u)
            return carry

        trips = (nchunk - _RING + 1) // _RING
        lax.fori_loop(0, trips, body, 0)
        for m in range(trips * _RING, nchunk):
            if m + _RING - 1 < nchunk:
                idx_issue(m + _RING - 1, (m + _RING - 1) % _RING)
            if m + _RING - 2 < nchunk:
                idx_wait(m + _RING - 2, (m + _RING - 2) % _RING)
                gather((m + _RING - 2) % _RING)
            drain_scatter(m % _RING)
        plsc.subcore_barrier()

        @pl.when(jnp.logical_and(core == 0, jnp.logical_not(is_last)))
        def _():
            pltpu.sync_copy(acc.at[pl.ds(stripe, _STRIPE)],
                            p0_hbm.at[pl.ds(stripe, _STRIPE)])

        @pl.when(jnp.logical_and(core == 0, is_last))
        def _():
            pltpu.sync_copy(acc.at[pl.ds(_LAST_BASE, _LAST_ROWS)],
                            p0_hbm.at[pl.ds(_LAST_BASE, _LAST_ROWS)])

        @pl.when(jnp.logical_and(core == 1, jnp.logical_not(is_last)))
        def _():
            pltpu.sync_copy(acc.at[pl.ds(stripe, _STRIPE)],
                            p1_hbm.at[pl.ds(stripe, _STRIPE)])

        @pl.when(jnp.logical_and(core == 1, is_last))
        def _():
            pltpu.sync_copy(acc.at[pl.ds(_LAST_BASE, _LAST_ROWS)],
                            p1_hbm.at[pl.ds(_LAST_BASE, _LAST_ROWS)])

    return agg


_agg1 = _make_agg(dual=False)
_agg2 = _make_agg(dual=True)

_BLK = 5000


def _tc1_body(ax, ah, h, wx, wh, b, u_out, hp_out):
    g = jnp.dot(ax[...], wx[...], preferred_element_type=jnp.float32)
    g = g + jnp.dot(ah[...], wh[...], preferred_element_type=jnp.float32)
    g = jax.nn.sigmoid(g + b[...])
    u_out[...] = g[:, _D:]
    hp_out[...] = g[:, :_D] * h[...]


def _tc1(ax, ah, h, wx, wh, b):
    row = pl.BlockSpec((_BLK, _D), lambda i: (i, 0))
    full = pl.BlockSpec((_D, 2 * _D), lambda i: (0, 0))
    bias = pl.BlockSpec((1, 2 * _D), lambda i: (0, 0))
    return pl.pallas_call(
        _tc1_body,
        grid=(_N // _BLK,),
        in_specs=[row, row, row, full, full, bias],
        out_specs=[row, row],
        out_shape=[
            jax.ShapeDtypeStruct((_N, _D), jnp.float32),
            jax.ShapeDtypeStruct((_N, _D), jnp.float32),
        ],
    )(ax, ah, h, wx, wh, b)


def _tc2_body(ax, ac0, ac1, u, h, wxc, whc, bc, out):
    acv = ac0[...] + ac1[...]
    c = jnp.dot(ax[...], wxc[...], preferred_element_type=jnp.float32)
    c = c + jnp.dot(acv, whc[...], preferred_element_type=jnp.float32)
    c = jax.nn.sigmoid(c + bc[...])
    uv = u[...]
    out[...] = uv * h[...] + (1.0 - uv) * c


def _tc2(ax, ac0, ac1, u, h, wxc, whc, bc):
    row = pl.BlockSpec((_BLK, _D), lambda i: (i, 0))
    full = pl.BlockSpec((_D, _D), lambda i: (0, 0))
    bias = pl.BlockSpec((1, _D), lambda i: (0, 0))
    return pl.pallas_call(
        _tc2_body,
        grid=(_N // _BLK,),
        in_specs=[row, row, row, row, row, full, full, bias],
        out_specs=row,
        out_shape=jax.ShapeDtypeStruct((_N, _D), jnp.float32),
    )(ax, ac0, ac1, u, h, wxc, whc, bc)


def kernel(x, hidden_states, edge_index, params):
    src1 = edge_index[0].reshape(_NW, _NCHUNK, 1, _CHUNK)
    dst1 = edge_index[1].reshape(_NW, _NCHUNK, 1, _CHUNK)
    src2 = edge_index[0].reshape(_NS, 2 * _NCHUNK, 1, _CHUNK)
    dst2 = edge_index[1].reshape(_NS, 2 * _NCHUNK, 1, _CHUNK)
    zeros = jnp.zeros((_LAST_ROWS, _D), jnp.float32)

    hiddens = []
    cur = x
    for l in range(_L):
        h = hidden_states[l]
        wr, wu, wc = params['W_r%d' % l], params['W_u%d' % l], params['W_c%d' % l]
        wx_ru = jnp.concatenate([wr[:_D], wu[:_D]], axis=1)
        wh_ru = jnp.concatenate([wr[_D:], wu[_D:]], axis=1)
        b_ru = jnp.concatenate([
            params['b_r%d' % l] + params['gb_r%d' % l],
            params['b_u%d' % l] + params['gb_u%d' % l],
        ])[None, :]
        wxc, whc = wc[:_D], wc[_D:]
        bc = (params['b_c%d' % l] + params['gb_c%d' % l])[None, :]

        ax, ah = _agg2(cur, h, src2, dst2, zeros)
        u, hp = _tc1(ax, ah, h, wx_ru, wh_ru, b_ru)
        ac0, ac1 = _agg1(hp, hp, src1, dst1, zeros)
        cur = _tc2(ax, ac0, ac1, u, h, wxc, whc, bc)
        hiddens.append(cur)
    return (cur, jnp.stack(hiddens))
